# Initial kernel scaffold; baseline (speedup 1.0000x reference)
import functools

import jax
import jax.numpy as jnp
from jax import lax
from jax.experimental import pallas as pl
from jax.experimental.pallas import tpu as pltpu
from jax.experimental.pallas import tpu_sc as plsc

N_NODES = 10000
N_EDGES = 320000
D_FEAT = 128
UNITS = 128

NC = 2
NS = 16
L = 16

EDGES_PER_CORE = N_EDGES // NC
EDGES_PER_TILE = EDGES_PER_CORE // NS
CHUNK = 128
N_FULL_CHUNKS = EDGES_PER_TILE // CHUNK
TAIL = EDGES_PER_TILE - N_FULL_CHUNKS * CHUNK
ROWS_PER_TILE = N_NODES // NS


def _sc_aggregate(x, src, dst, ew, zeros):
    mesh = plsc.VectorSubcoreMesh(core_axis_name="c", subcore_axis_name="s")

    @functools.partial(
        pl.kernel,
        out_type=jax.ShapeDtypeStruct((NC, N_NODES, D_FEAT), jnp.float32),
        mesh=mesh,
        scratch_types=[
            pltpu.VMEM((CHUNK,), jnp.int32),
            pltpu.VMEM((CHUNK,), jnp.int32),
            pltpu.VMEM((CHUNK,), jnp.float32),
            pltpu.VMEM((CHUNK, D_FEAT), jnp.float32),
            pltpu.VMEM_SHARED((N_NODES, D_FEAT), jnp.float32),
        ],
    )
    def k(x_hbm, src_hbm, dst_hbm, ew_hbm, zeros_hbm, out_hbm,
          src_v, dst_v, w_v, rows_v, agg_sh):
        cid = lax.axis_index("c")
        sid = lax.axis_index("s")
        tbase = cid * EDGES_PER_CORE + sid * EDGES_PER_TILE

        r0 = sid * ROWS_PER_TILE
        pltpu.sync_copy(zeros_hbm.at[pl.ds(r0, ROWS_PER_TILE)],
                        agg_sh.at[pl.ds(r0, ROWS_PER_TILE)])
        plsc.subcore_barrier()

        def do_chunk(base, n):
            pltpu.sync_copy(src_hbm.at[pl.ds(base, n)], src_v.at[pl.ds(0, n)])
            pltpu.sync_copy(dst_hbm.at[pl.ds(base, n)], dst_v.at[pl.ds(0, n)])
            pltpu.sync_copy(ew_hbm.at[pl.ds(base, n)], w_v.at[pl.ds(0, n)])
            if n == CHUNK:
                pltpu.sync_copy(x_hbm.at[src_v], rows_v)
            else:
                pltpu.sync_copy(x_hbm.at[src_v.at[pl.ds(0, n)]],
                                rows_v.at[pl.ds(0, n)])

            def scale_body(e, carry):
                wv = plsc.load_gather(w_v, [jnp.full((L,), e, jnp.int32)])
                for f in range(D_FEAT // L):
                    sl = pl.ds(f * L, L)
                    rows_v[e, sl] = rows_v[e, sl] * wv
                return carry

            lax.fori_loop(0, n, scale_body, 0)
            if n == CHUNK:
                pltpu.sync_copy(rows_v, agg_sh.at[dst_v], add=True)
            else:
                pltpu.sync_copy(rows_v.at[pl.ds(0, n)],
                                agg_sh.at[dst_v.at[pl.ds(0, n)]], add=True)

        def chunk_body(i, carry):
            do_chunk(tbase + i * CHUNK, CHUNK)
            return carry

        lax.fori_loop(0, N_FULL_CHUNKS, chunk_body, 0)
        if TAIL:
            do_chunk(tbase + N_FULL_CHUNKS * CHUNK, TAIL)

        plsc.subcore_barrier()
        pltpu.sync_copy(agg_sh.at[pl.ds(r0, ROWS_PER_TILE)],
                        out_hbm.at[cid, pl.ds(r0, ROWS_PER_TILE)])

    return k(x, src, dst, ew, zeros)


def _tc_finish(partials, w, bias2d):
    BLK = 1000

    def body(p_ref, w_ref, b_ref, o_ref):
        p = p_ref[0] + p_ref[1]
        acc = jnp.dot(p, w_ref[...], preferred_element_type=jnp.float32)
        o_ref[...] = jnp.maximum(acc + b_ref[...], 0.0)

    return pl.pallas_call(
        body,
        grid=(N_NODES // BLK,),
        in_specs=[
            pl.BlockSpec((NC, BLK, D_FEAT), lambda i: (0, i, 0)),
            pl.BlockSpec((D_FEAT, UNITS), lambda i: (0, 0)),
            pl.BlockSpec((1, UNITS), lambda i: (0, 0)),
        ],
        out_specs=pl.BlockSpec((BLK, UNITS), lambda i: (i, 0)),
        out_shape=jax.ShapeDtypeStruct((N_NODES, UNITS), jnp.float32),
    )(partials, w, bias2d)


@jax.jit
def kernel(x, edge_index, edge_weight, kernel, bias):
    src = edge_index[0]
    dst = edge_index[1]
    zeros = jnp.zeros((N_NODES, D_FEAT), jnp.float32)
    partials = _sc_aggregate(x, src, dst, edge_weight, zeros)
    return _tc_finish(partials, kernel, bias.reshape(1, UNITS))


# speedup vs baseline: 5.0422x; 5.0422x over previous
import functools

import jax
import jax.numpy as jnp
from jax import lax
from jax.experimental import pallas as pl
from jax.experimental.pallas import tpu as pltpu
from jax.experimental.pallas import tpu_sc as plsc

N_NODES = 10000
N_EDGES = 320000
D_FEAT = 128
UNITS = 128

NC = 2
NS = 16
L = 16

EDGES_PER_CORE = N_EDGES // NC
EDGES_PER_TILE = EDGES_PER_CORE // NS
CHUNK = 128
N_FULL_CHUNKS = EDGES_PER_TILE // CHUNK
TAIL = EDGES_PER_TILE - N_FULL_CHUNKS * CHUNK
ROWS_PER_TILE = 640
LAST_ROW_BASE = N_NODES - ROWS_PER_TILE


def _sc_aggregate(x, src, dst, ew, zeros):
    mesh = plsc.VectorSubcoreMesh(core_axis_name="c", subcore_axis_name="s")

    @functools.partial(
        pl.kernel,
        out_type=jax.ShapeDtypeStruct((NC, N_NODES, D_FEAT), jnp.float32),
        mesh=mesh,
        scratch_types=[
            pltpu.VMEM((CHUNK,), jnp.int32),
            pltpu.VMEM((CHUNK,), jnp.int32),
            pltpu.VMEM((CHUNK,), jnp.float32),
            pltpu.VMEM((CHUNK, D_FEAT), jnp.float32),
            pltpu.VMEM_SHARED((N_NODES, D_FEAT), jnp.float32),
        ],
    )
    def k(x_hbm, src_hbm, dst_hbm, ew_hbm, zeros_hbm, out_hbm,
          src_v, dst_v, w_v, rows_v, agg_sh):
        cid = lax.axis_index("c")
        sid = lax.axis_index("s")
        tbase = cid * EDGES_PER_CORE + sid * EDGES_PER_TILE

        r0 = jnp.minimum(sid * ROWS_PER_TILE, LAST_ROW_BASE)
        pltpu.sync_copy(zeros_hbm.at[pl.ds(r0, ROWS_PER_TILE)],
                        agg_sh.at[pl.ds(r0, ROWS_PER_TILE)])
        plsc.subcore_barrier()

        def do_chunk(base, n):
            pltpu.sync_copy(src_hbm.at[pl.ds(base, n)], src_v.at[pl.ds(0, n)])
            pltpu.sync_copy(dst_hbm.at[pl.ds(base, n)], dst_v.at[pl.ds(0, n)])
            pltpu.sync_copy(ew_hbm.at[pl.ds(base, n)], w_v.at[pl.ds(0, n)])
            if n == CHUNK:
                pltpu.sync_copy(x_hbm.at[src_v], rows_v)
            else:
                pltpu.sync_copy(x_hbm.at[src_v.at[pl.ds(0, n)]],
                                rows_v.at[pl.ds(0, n)])

            def scale_body(g, carry):
                wgrp = w_v[pl.ds(g * L, L)]
                for lane in range(L):
                    e = g * L + lane
                    wv = wgrp.at[jnp.full((L,), lane, jnp.int32)].get(
                        mode="promise_in_bounds")
                    for f in range(D_FEAT // L):
                        sl = pl.ds(f * L, L)
                        rows_v[e, sl] = rows_v[e, sl] * wv
                return carry

            lax.fori_loop(0, n // L, scale_body, 0)
            if n == CHUNK:
                pltpu.sync_copy(rows_v, agg_sh.at[dst_v], add=True)
            else:
                pltpu.sync_copy(rows_v.at[pl.ds(0, n)],
                                agg_sh.at[dst_v.at[pl.ds(0, n)]], add=True)

        def chunk_body(i, carry):
            do_chunk(tbase + i * CHUNK, CHUNK)
            return carry

        lax.fori_loop(0, N_FULL_CHUNKS, chunk_body, 0)
        if TAIL:
            do_chunk(tbase + N_FULL_CHUNKS * CHUNK, TAIL)

        plsc.subcore_barrier()
        pltpu.sync_copy(agg_sh.at[pl.ds(r0, ROWS_PER_TILE)],
                        out_hbm.at[cid, pl.ds(r0, ROWS_PER_TILE)])

    return k(x, src, dst, ew, zeros)


def _tc_finish(partials, w, bias2d):
    BLK = 1000

    def body(p_ref, w_ref, b_ref, o_ref):
        p = p_ref[0] + p_ref[1]
        acc = jnp.dot(p, w_ref[...], preferred_element_type=jnp.float32)
        o_ref[...] = jnp.maximum(acc + b_ref[...], 0.0)

    return pl.pallas_call(
        body,
        grid=(N_NODES // BLK,),
        in_specs=[
            pl.BlockSpec((NC, BLK, D_FEAT), lambda i: (0, i, 0)),
            pl.BlockSpec((D_FEAT, UNITS), lambda i: (0, 0)),
            pl.BlockSpec((1, UNITS), lambda i: (0, 0)),
        ],
        out_specs=pl.BlockSpec((BLK, UNITS), lambda i: (i, 0)),
        out_shape=jax.ShapeDtypeStruct((N_NODES, UNITS), jnp.float32),
    )(partials, w, bias2d)


@jax.jit
def kernel(x, edge_index, edge_weight, kernel, bias):
    src = edge_index[0]
    dst = edge_index[1]
    zeros = jnp.zeros((N_NODES, D_FEAT), jnp.float32)
    partials = _sc_aggregate(x, src, dst, edge_weight, zeros)
    return _tc_finish(partials, kernel, bias.reshape(1, UNITS))
